# baseline (speedup 1.0000x reference)
import jax
import jax.numpy as jnp
from jax.experimental import pallas as pl


def _lap(x, src, dst, w, n):
    return jax.ops.segment_sum(w[:, None] * jnp.take(x, src, axis=0), dst, num_segments=n)


def _cheb(x, src, dst, w, W, b):
    n = x.shape[0]
    Tx0 = x
    out = Tx0 @ W[0]
    Tx1 = _lap(Tx0, src, dst, w, n)
    out = out + Tx1 @ W[1]
    for k in range(2, W.shape[0]):
        Tx2 = 2.0 * _lap(Tx1, src, dst, w, n) - Tx0
        out = out + Tx2 @ W[k]
        Tx0, Tx1 = Tx1, Tx2
    return out + b


def _bn(x, g, be):
    m = jnp.mean(x, axis=0)
    v = jnp.var(x, axis=0)
    return g * (x - m) / jnp.sqrt(v + 1e-5) + be


def kernel(x, edge_index, W1, b1, g1, be1, W2, b2, g2, be2, W3, b3):
    src = edge_index[0]
    dst = edge_index[1]
    n = x.shape[0]
    deg = jax.ops.segment_sum(jnp.ones((src.shape[0],), jnp.float32), src, num_segments=n)
    dis = jnp.where(deg > 0, 1.0 / jnp.sqrt(jnp.maximum(deg, 1e-12)), 0.0)
    w = -dis[src] * dis[dst]
    h = jax.nn.relu(_bn(_cheb(x, src, dst, w, W1, b1), g1, be1))
    h = jax.nn.relu(_bn(_cheb(h, src, dst, w, W2, b2), g2, be2))
    return _cheb(h, src, dst, w, W3, b3)


# speedup vs baseline: 14.0114x; 14.0114x over previous
import functools

import jax
import jax.numpy as jnp
from jax import lax
from jax.experimental import pallas as pl
from jax.experimental.pallas import tpu as pltpu
from jax.experimental.pallas import tpu_sc as plsc

_N = 10000
_E = 320000
_NSC = 2
_NSUB = 16
_NW = _NSC * _NSUB
_EPW = _E // _NW
_CH = 125
_NCHUNK = _EPW // _CH
_RPS = _N // _NSUB


@functools.lru_cache(maxsize=None)
def _make_spmm(fp: int):
    mesh = plsc.VectorSubcoreMesh(
        core_axis_name="c", subcore_axis_name="s",
        num_cores=_NSC, num_subcores=_NSUB)

    @functools.partial(
        pl.kernel,
        out_type=jax.ShapeDtypeStruct((_NSC, _N, fp), jnp.float32),
        mesh=mesh,
        scratch_types=[
            pltpu.VMEM((_NCHUNK, _CH), jnp.int32),
            pltpu.VMEM((_NCHUNK, _CH), jnp.int32),
            pltpu.VMEM((_CH, fp), jnp.float32),
            pltpu.VMEM_SHARED((_N, fp), jnp.float32),
            pltpu.SemaphoreType.DMA,
        ],
        compiler_params=pltpu.CompilerParams(use_tc_tiling_on_sc=False),
    )
    def spmm(u_hbm, gidx_hbm, sidx_hbm, zero_hbm, out_hbm,
             gi_v, si_v, rows_v, acc, sem):
        cid = lax.axis_index("c")
        sid = lax.axis_index("s")
        wid = cid * _NSUB + sid
        pltpu.sync_copy(zero_hbm.at[pl.ds(sid * _RPS, _RPS)],
                        acc.at[pl.ds(sid * _RPS, _RPS)])
        pltpu.sync_copy(gidx_hbm.at[pl.ds(wid * _NCHUNK, _NCHUNK)], gi_v)
        pltpu.sync_copy(sidx_hbm.at[pl.ds(wid * _NCHUNK, _NCHUNK)], si_v)
        plsc.subcore_barrier()

        def body(i, _):
            pltpu.async_copy(u_hbm.at[gi_v.at[i]], rows_v, sem).wait()
            pltpu.sync_copy(rows_v, acc.at[si_v.at[i]], add=True)
            return 0

        lax.fori_loop(0, _NCHUNK, body, 0)
        plsc.subcore_barrier()
        pltpu.sync_copy(acc.at[pl.ds(sid * _RPS, _RPS)],
                        out_hbm.at[cid, pl.ds(sid * _RPS, _RPS)])

    return spmm


def _spmm(u, gidx2d, sidx2d, fp):
    zero = jnp.zeros((_N, fp), jnp.float32)
    parts = _make_spmm(fp)(u, gidx2d, sidx2d, zero)
    return parts[0] + parts[1]


def _proj(h, w):
    K, Fin, Fout = w.shape

    def body(h_ref, w_ref, o_ref):
        hv = h_ref[...]
        for k in range(K):
            o_ref[k, ...] = jnp.dot(hv, w_ref[k], preferred_element_type=jnp.float32)

    return pl.pallas_call(
        body,
        out_shape=jax.ShapeDtypeStruct((K, h.shape[0], Fout), jnp.float32),
    )(h, w)


def _cheb(h, dis, gidx2d, sidx2d, w_pad, fp):
    K = w_pad.shape[0]
    Z = _proj(h, w_pad)
    disc = dis[:, None]

    def S(v):
        g = _spmm(disc * v, gidx2d, sidx2d, fp)
        return -disc * g

    bk1 = Z[K - 1]
    bk2 = jnp.zeros_like(bk1)
    for k in range(K - 2, 0, -1):
        bk1, bk2 = Z[k] + 2.0 * S(bk1) - bk2, bk1
    return Z[0] + S(bk1) - bk2


def _bn_relu(x, g, be):
    m = jnp.mean(x, axis=0)
    v = jnp.var(x, axis=0)
    return jax.nn.relu(g * (x - m) / jnp.sqrt(v + 1e-5) + be)


def _pad_w(w, fout_pad):
    K, fin, fout = w.shape
    return jnp.concatenate(
        [w, jnp.zeros((K, fin, fout_pad - fout), jnp.float32)], axis=-1)


def kernel(x, edge_index, W1, b1, g1, be1, W2, b2, g2, be2, W3, b3):
    src = edge_index[0]
    dst = edge_index[1]
    gidx2d = src.reshape(_E // _CH, _CH)
    sidx2d = dst.reshape(_E // _CH, _CH)
    srcidx2d = gidx2d

    ones = jnp.ones((_N, 16), jnp.float32)
    deg = _spmm(ones, srcidx2d, srcidx2d, 16)[:, 0]
    dis = jnp.where(deg > 0, 1.0 / jnp.sqrt(jnp.maximum(deg, 1e-12)), 0.0)

    h = _cheb(x, dis, gidx2d, sidx2d, W1, 64) + b1
    h = _bn_relu(h, g1, be1)

    h = _cheb(h, dis, gidx2d, sidx2d, _pad_w(W2, 32), 32)[:, :18] + b2
    h = _bn_relu(h, g2, be2)

    out = _cheb(h, dis, gidx2d, sidx2d, _pad_w(W3, 16), 16)[:, :10] + b3
    return out


# speedup vs baseline: 24.1804x; 1.7258x over previous
import functools

import jax
import jax.numpy as jnp
from jax import lax
from jax.experimental import pallas as pl
from jax.experimental.pallas import tpu as pltpu
from jax.experimental.pallas import tpu_sc as plsc

_N = 10000
_E = 320000
_NSC = 2
_NSUB = 16
_NW = _NSC * _NSUB
_EPW = _E // _NW
_CH = 125
_NCHUNK = _EPW // _CH
_RPS = _N // _NSUB


def _cr(fp: int) -> int:
    return 4 if fp > 32 else 8


@functools.lru_cache(maxsize=None)
def _make_spmm(fp: int):
    mesh = plsc.VectorSubcoreMesh(
        core_axis_name="c", subcore_axis_name="s",
        num_cores=_NSC, num_subcores=_NSUB)
    cr = _cr(fp)
    nch = _EPW // (cr * _CH)
    nch2 = nch // 2

    @functools.partial(
        pl.kernel,
        out_type=jax.ShapeDtypeStruct((_NSC, _N, fp), jnp.float32),
        mesh=mesh,
        scratch_types=[
            pltpu.VMEM((nch, cr * _CH), jnp.int32),
            pltpu.VMEM((nch, cr * _CH), jnp.int32),
            pltpu.VMEM((cr * _CH, fp), jnp.float32),
            pltpu.VMEM((cr * _CH, fp), jnp.float32),
            pltpu.VMEM_SHARED((_N, fp), jnp.float32),
            pltpu.SemaphoreType.DMA,
            pltpu.SemaphoreType.DMA,
        ],
        compiler_params=pltpu.CompilerParams(use_tc_tiling_on_sc=False),
    )
    def spmm(u_hbm, gidx_hbm, sidx_hbm, zero_hbm, out_hbm,
             gi_v, si_v, rows0, rows1, acc, sem0, sem1):
        cid = lax.axis_index("c")
        sid = lax.axis_index("s")
        wid = cid * _NSUB + sid
        pltpu.sync_copy(zero_hbm.at[pl.ds(sid * _RPS, _RPS)],
                        acc.at[pl.ds(sid * _RPS, _RPS)])
        pltpu.sync_copy(gidx_hbm.at[pl.ds(wid * nch, nch)], gi_v)
        pltpu.sync_copy(sidx_hbm.at[pl.ds(wid * nch, nch)], si_v)
        plsc.subcore_barrier()

        pltpu.async_copy(u_hbm.at[gi_v.at[0]], rows0, sem0)

        def body(i2, _):
            a = 2 * i2
            b = a + 1
            pltpu.make_async_copy(u_hbm.at[gi_v.at[a]], rows0, sem0).wait()
            pltpu.async_copy(u_hbm.at[gi_v.at[b]], rows1, sem1)
            pltpu.sync_copy(rows0, acc.at[si_v.at[a]], add=True)
            pltpu.make_async_copy(u_hbm.at[gi_v.at[b]], rows1, sem1).wait()

            @pl.when(i2 + 1 < nch2)
            def _():
                pltpu.async_copy(u_hbm.at[gi_v.at[b + 1]], rows0, sem0)

            pltpu.sync_copy(rows1, acc.at[si_v.at[b]], add=True)
            return 0

        lax.fori_loop(0, nch2, body, 0)
        plsc.subcore_barrier()
        pltpu.sync_copy(acc.at[pl.ds(sid * _RPS, _RPS)],
                        out_hbm.at[cid, pl.ds(sid * _RPS, _RPS)])

    return spmm


def _spmm(u, gidx, sidx, fp):
    cr = _cr(fp)
    zero = jnp.zeros((_N, fp), jnp.float32)
    parts = _make_spmm(fp)(u, gidx.reshape(-1, cr * _CH), sidx.reshape(-1, cr * _CH), zero)
    return parts[0] + parts[1]


def _proj(h, w):
    K, Fin, Fout = w.shape

    def body(h_ref, w_ref, o_ref):
        hv = h_ref[...]
        for k in range(K):
            o_ref[k, ...] = jnp.dot(hv, w_ref[k], preferred_element_type=jnp.float32)

    return pl.pallas_call(
        body,
        out_shape=jax.ShapeDtypeStruct((K, h.shape[0], Fout), jnp.float32),
    )(h, w)


def _cheb(h, dis, gidx2d, sidx2d, w_pad, fp):
    K = w_pad.shape[0]
    Z = _proj(h, w_pad)
    disc = dis[:, None]

    def S(v):
        g = _spmm(disc * v, gidx2d, sidx2d, fp)
        return -disc * g

    bk1 = Z[K - 1]
    bk2 = jnp.zeros_like(bk1)
    for k in range(K - 2, 0, -1):
        bk1, bk2 = Z[k] + 2.0 * S(bk1) - bk2, bk1
    return Z[0] + S(bk1) - bk2


def _bn_relu(x, g, be):
    m = jnp.mean(x, axis=0)
    v = jnp.var(x, axis=0)
    return jax.nn.relu(g * (x - m) / jnp.sqrt(v + 1e-5) + be)


def _pad_w(w, fout_pad):
    K, fin, fout = w.shape
    return jnp.concatenate(
        [w, jnp.zeros((K, fin, fout_pad - fout), jnp.float32)], axis=-1)


def kernel(x, edge_index, W1, b1, g1, be1, W2, b2, g2, be2, W3, b3):
    src = edge_index[0]
    dst = edge_index[1]
    gidx2d = src.reshape(_E // _CH, _CH)
    sidx2d = dst.reshape(_E // _CH, _CH)
    srcidx2d = gidx2d

    ones = jnp.ones((_N, 16), jnp.float32)
    deg = _spmm(ones, srcidx2d, srcidx2d, 16)[:, 0]
    dis = jnp.where(deg > 0, 1.0 / jnp.sqrt(jnp.maximum(deg, 1e-12)), 0.0)

    h = _cheb(x, dis, gidx2d, sidx2d, W1, 64) + b1
    h = _bn_relu(h, g1, be1)

    h = _cheb(h, dis, gidx2d, sidx2d, _pad_w(W2, 32), 32)[:, :18] + b2
    h = _bn_relu(h, g2, be2)

    out = _cheb(h, dis, gidx2d, sidx2d, _pad_w(W3, 16), 16)[:, :10] + b3
    return out


# speedup vs baseline: 24.5466x; 1.0151x over previous
import functools

import jax
import jax.numpy as jnp
from jax import lax
from jax.experimental import pallas as pl
from jax.experimental.pallas import tpu as pltpu
from jax.experimental.pallas import tpu_sc as plsc

_N = 10000
_E = 320000
_NSC = 2
_NSUB = 16
_NW = _NSC * _NSUB
_EPW = _E // _NW
_CH = 125
_NCHUNK = _EPW // _CH
_RPS = _N // _NSUB


def _cr(fp: int) -> int:
    if fp > 32:
        return 4
    return 10 if fp > 16 else 20


@functools.lru_cache(maxsize=None)
def _make_spmm(fp: int):
    mesh = plsc.VectorSubcoreMesh(
        core_axis_name="c", subcore_axis_name="s",
        num_cores=_NSC, num_subcores=_NSUB)
    cr = _cr(fp)
    nch = _EPW // (cr * _CH)
    nch2 = nch // 2

    @functools.partial(
        pl.kernel,
        out_type=jax.ShapeDtypeStruct((_NSC, _N, fp), jnp.float32),
        mesh=mesh,
        scratch_types=[
            pltpu.VMEM((2, nch, cr * _CH), jnp.int32),
            pltpu.VMEM((cr * _CH, fp), jnp.float32),
            pltpu.VMEM((cr * _CH, fp), jnp.float32),
            pltpu.VMEM_SHARED((_N, fp), jnp.float32),
            pltpu.SemaphoreType.DMA,
            pltpu.SemaphoreType.DMA,
        ],
        compiler_params=pltpu.CompilerParams(use_tc_tiling_on_sc=False),
    )
    def spmm(u_hbm, idx_hbm, zero_hbm, out_hbm,
             idx_v, rows0, rows1, acc, sem0, sem1):
        cid = lax.axis_index("c")
        sid = lax.axis_index("s")
        wid = cid * _NSUB + sid
        pltpu.sync_copy(zero_hbm.at[pl.ds(sid * _RPS, _RPS)],
                        acc.at[pl.ds(sid * _RPS, _RPS)])
        pltpu.sync_copy(idx_hbm.at[:, pl.ds(wid * nch, nch)], idx_v)
        plsc.subcore_barrier()

        pltpu.async_copy(u_hbm.at[idx_v.at[0, 0]], rows0, sem0)

        def body(i2, _):
            a = 2 * i2
            b = a + 1
            pltpu.make_async_copy(u_hbm.at[idx_v.at[0, a]], rows0, sem0).wait()
            pltpu.async_copy(u_hbm.at[idx_v.at[0, b]], rows1, sem1)
            pltpu.sync_copy(rows0, acc.at[idx_v.at[1, a]], add=True)
            pltpu.make_async_copy(u_hbm.at[idx_v.at[0, b]], rows1, sem1).wait()

            @pl.when(i2 + 1 < nch2)
            def _():
                pltpu.async_copy(u_hbm.at[idx_v.at[0, b + 1]], rows0, sem0)

            pltpu.sync_copy(rows1, acc.at[idx_v.at[1, b]], add=True)
            return 0

        lax.fori_loop(0, nch2, body, 0)
        plsc.subcore_barrier()
        pltpu.sync_copy(acc.at[pl.ds(sid * _RPS, _RPS)],
                        out_hbm.at[cid, pl.ds(sid * _RPS, _RPS)])

    return spmm


def _spmm(u, idx, fp):
    cr = _cr(fp)
    zero = jnp.zeros((_N, fp), jnp.float32)
    parts = _make_spmm(fp)(u, idx.reshape(2, -1, cr * _CH), zero)
    return parts[0] + parts[1]


def _proj(h, w):
    K, Fin, Fout = w.shape

    def body(h_ref, w_ref, o_ref):
        hv = h_ref[...]
        for k in range(K):
            o_ref[k, ...] = jnp.dot(hv, w_ref[k], preferred_element_type=jnp.float32)

    return pl.pallas_call(
        body,
        out_shape=jax.ShapeDtypeStruct((K, h.shape[0], Fout), jnp.float32),
    )(h, w)


def _cheb(h, dis, idx, w_pad, fp):
    K = w_pad.shape[0]
    Z = _proj(h, w_pad)
    disc = dis[:, None]

    def S(v):
        g = _spmm(disc * v, idx, fp)
        return -disc * g

    bk1 = Z[K - 1]
    bk2 = jnp.zeros_like(bk1)
    for k in range(K - 2, 0, -1):
        bk1, bk2 = Z[k] + 2.0 * S(bk1) - bk2, bk1
    return Z[0] + S(bk1) - bk2


def _bn_relu(x, g, be):
    m = jnp.mean(x, axis=0)
    v = jnp.var(x, axis=0)
    return jax.nn.relu(g * (x - m) / jnp.sqrt(v + 1e-5) + be)


def _pad_w(w, fout_pad):
    K, fin, fout = w.shape
    return jnp.concatenate(
        [w, jnp.zeros((K, fin, fout_pad - fout), jnp.float32)], axis=-1)


def kernel(x, edge_index, W1, b1, g1, be1, W2, b2, g2, be2, W3, b3):
    src = edge_index[0]

    ones = jnp.ones((_N, 8), jnp.float32)
    deg = _spmm(ones, jnp.stack([src, src]), 8)[:, 0]
    dis = jnp.where(deg > 0, 1.0 / jnp.sqrt(jnp.maximum(deg, 1e-12)), 0.0)

    h = _cheb(x, dis, edge_index, W1, 64) + b1
    h = _bn_relu(h, g1, be1)

    h = _cheb(h, dis, edge_index, _pad_w(W2, 24), 24)[:, :18] + b2
    h = _bn_relu(h, g2, be2)

    out = _cheb(h, dis, edge_index, _pad_w(W3, 16), 16)[:, :10] + b3
    return out
